# baseline (speedup 1.0000x reference)
import functools

import jax
import jax.numpy as jnp
from jax import lax
from jax.experimental import pallas as pl
from jax.experimental.pallas import tpu as pltpu
from jax.experimental.pallas import tpu_sc as plsc

_L = 16
_W = 128
_CHUNK = 128


def _scores_sc(user_ids, item_ids, up0, up1, ip0, ip1):
    B = user_ids.shape[0]
    D = _W // 2
    nu_half = up0.shape[0]
    ni_half = ip0.shape[0]
    info = plsc.get_sparse_core_info()
    nw = info.num_cores * info.num_subcores
    b_per_w = B // nw
    n_chunks = b_per_w // _CHUNK

    mesh = plsc.VectorSubcoreMesh(core_axis_name="c", subcore_axis_name="s")

    @functools.partial(
        pl.kernel,
        mesh=mesh,
        compiler_params=pltpu.CompilerParams(needs_layout_passes=False),
        out_type=jax.ShapeDtypeStruct((B,), jnp.float32),
        scratch_types=[
            pltpu.VMEM((b_per_w,), jnp.int32),
            pltpu.VMEM((b_per_w,), jnp.int32),
            pltpu.VMEM((_CHUNK,), jnp.int32),
            pltpu.VMEM((_CHUNK,), jnp.int32),
            pltpu.VMEM((_CHUNK,), jnp.int32),
            pltpu.VMEM((_CHUNK,), jnp.int32),
            pltpu.VMEM((_CHUNK, _W), jnp.float32),
            pltpu.VMEM((_CHUNK, _W), jnp.float32),
            pltpu.VMEM((_CHUNK, _W), jnp.float32),
            pltpu.VMEM((_CHUNK, _W), jnp.float32),
            pltpu.VMEM((b_per_w,), jnp.float32),
            pltpu.SemaphoreType.DMA,
            pltpu.SemaphoreType.DMA,
        ],
    )
    def k(uids_hbm, iids_hbm, up0_hbm, up1_hbm, ip0_hbm, ip1_hbm, out_hbm,
          uid_v, iid_v, ux0, ux1, ix0, ix1, ub0, ub1, ib0, ib1, out_v,
          sem_u, sem_i):
        wid = lax.axis_index("s") * info.num_cores + lax.axis_index("c")
        base = wid * b_per_w
        pltpu.sync_copy(uids_hbm.at[pl.ds(base, b_per_w)], uid_v)
        pltpu.sync_copy(iids_hbm.at[pl.ds(base, b_per_w)], iid_v)

        lane = lax.iota(jnp.int32, _L)

        def chunk_body(g, carry):
            g0 = g * _CHUNK
            for j in range(_CHUNK // _L):
                s = pl.ds(g0 + j * _L, _L)
                d_ = pl.ds(j * _L, _L)
                up = uid_v[s] >> 1
                ip_ = iid_v[s] >> 1
                ux0[d_] = jnp.minimum(up, nu_half - 1)
                ux1[d_] = jnp.clip(up - nu_half, 0, nu_half - 1)
                ix0[d_] = jnp.minimum(ip_, ni_half - 1)
                ix1[d_] = jnp.clip(ip_ - ni_half, 0, ni_half - 1)
            copies = [
                pltpu.async_copy(up0_hbm.at[ux0], ub0, sem_u),
                pltpu.async_copy(up1_hbm.at[ux1], ub1, sem_u),
                pltpu.async_copy(ip0_hbm.at[ix0], ib0, sem_i),
                pltpu.async_copy(ip1_hbm.at[ix1], ib1, sem_i),
            ]
            for c in copies:
                c.wait()

            def blk_body(blk, carry2):
                r0 = g0 + blk * _L
                rows = blk * _L + lane
                uids = uid_v[pl.ds(r0, _L)]
                iids = iid_v[pl.ds(r0, _L)]
                umask = (uids >> 1) >= nu_half
                imask = (iids >> 1) >= ni_half
                uphase = (uids & 1) * D
                iphase = (iids & 1) * D
                acc = jnp.zeros((_L,), jnp.float32)
                for d in range(D):
                    ucol = uphase + d
                    icol = iphase + d
                    u0 = plsc.load_gather(ub0, [rows, ucol])
                    u1 = plsc.load_gather(ub1, [rows, ucol])
                    v0 = plsc.load_gather(ib0, [rows, icol])
                    v1 = plsc.load_gather(ib1, [rows, icol])
                    u = jnp.where(umask, u1, u0)
                    v = jnp.where(imask, v1, v0)
                    acc = acc + u * v
                out_v[pl.ds(r0, _L)] = acc
                return carry2

            lax.fori_loop(0, _CHUNK // _L, blk_body, 0)
            return carry

        lax.fori_loop(0, n_chunks, chunk_body, 0)
        pltpu.sync_copy(out_v, out_hbm.at[pl.ds(base, b_per_w)])

    return k(user_ids, item_ids, up0, up1, ip0, ip1)


def kernel(user_ids, item_ids, user_table, item_table):
    B = user_ids.shape[0]
    nu = user_table.shape[0]
    ni = item_table.shape[0]
    up0 = user_table[:nu // 2].reshape(-1, _W)
    up1 = user_table[nu // 2:].reshape(-1, _W)
    ip0 = item_table[:ni // 2].reshape(-1, _W)
    ip1 = item_table[ni // 2:].reshape(-1, _W)
    scores = _scores_sc(user_ids.astype(jnp.int32), item_ids.astype(jnp.int32),
                        up0, up1, ip0, ip1)
    return scores.reshape(B, 1)
